# baseline (speedup 1.0000x reference)
import jax
import jax.numpy as jnp
from jax.experimental import pallas as pl
from jax.experimental.pallas import tpu as pltpu

NUM_EXPERTS = 64
TOP_K = 8
BLOCK_ROWS = 1024


def _router_block(x_ref, w_ref, b_ref, out_ref):
    x = x_ref[...]
    logits = jnp.dot(x, w_ref[...], preferred_element_type=jnp.float32)
    logits = logits + b_ref[...][None, :]

    m = jnp.max(logits, axis=1, keepdims=True)
    hit = logits == m
    acc = jnp.where(hit, 1.0, 0.0)
    work = jnp.where(hit, -jnp.inf, logits)
    for _ in range(TOP_K - 1):
        cur = jnp.max(work, axis=1, keepdims=True)
        hit = work == cur
        acc = jnp.where(hit, jnp.exp(cur - m), acc)
        work = jnp.where(hit, -jnp.inf, work)

    denom = jnp.sum(acc, axis=1, keepdims=True)
    out_ref[...] = acc / (denom * TOP_K)


@jax.jit
def kernel(hidden_states, W, b):
    n_tokens = hidden_states.shape[0]
    d_model = hidden_states.shape[1]
    grid = (n_tokens // BLOCK_ROWS,)
    return pl.pallas_call(
        _router_block,
        grid=grid,
        in_specs=[
            pl.BlockSpec((BLOCK_ROWS, d_model), lambda i: (i, 0)),
            pl.BlockSpec((d_model, NUM_EXPERTS), lambda i: (0, 0)),
            pl.BlockSpec((NUM_EXPERTS,), lambda i: (0,)),
        ],
        out_specs=pl.BlockSpec((BLOCK_ROWS, NUM_EXPERTS), lambda i: (i, 0)),
        out_shape=jax.ShapeDtypeStruct((n_tokens, NUM_EXPERTS), jnp.float32),
        compiler_params=pltpu.CompilerParams(
            dimension_semantics=("parallel",),
        ),
    )(hidden_states.astype(jnp.float32), W, b)


# speedup vs baseline: 1.0028x; 1.0028x over previous
import jax
import jax.numpy as jnp
from jax.experimental import pallas as pl
from jax.experimental.pallas import tpu as pltpu

NUM_EXPERTS = 64
TOP_K = 8
BLOCK_ROWS = 1024


def _router_block(x_ref, w_ref, b_ref, out_ref):
    x = x_ref[...]
    logits = jnp.dot(x, w_ref[...], preferred_element_type=jnp.float32)
    logits = logits + b_ref[...][None, :]

    m = jnp.max(logits, axis=1, keepdims=True)
    hit = logits == m
    acc = jnp.where(hit, 1.0, 0.0)
    work = jnp.where(hit, -jnp.inf, logits)
    for _ in range(TOP_K - 1):
        cur = jnp.max(work, axis=1, keepdims=True)
        hit = work == cur
        acc = acc + jnp.where(hit, jnp.exp(cur - m), 0.0)
        work = jnp.where(hit, -jnp.inf, work)

    denom = jnp.sum(acc, axis=1, keepdims=True)
    out_ref[...] = acc / (denom * TOP_K)


@jax.jit
def kernel(hidden_states, W, b):
    n_tokens = hidden_states.shape[0]
    d_model = hidden_states.shape[1]
    grid = (n_tokens // BLOCK_ROWS,)
    return pl.pallas_call(
        _router_block,
        grid=grid,
        in_specs=[
            pl.BlockSpec((BLOCK_ROWS, d_model), lambda i: (i, 0)),
            pl.BlockSpec((d_model, NUM_EXPERTS), lambda i: (0, 0)),
            pl.BlockSpec((NUM_EXPERTS,), lambda i: (0,)),
        ],
        out_specs=pl.BlockSpec((BLOCK_ROWS, NUM_EXPERTS), lambda i: (i, 0)),
        out_shape=jax.ShapeDtypeStruct((n_tokens, NUM_EXPERTS), jnp.float32),
        compiler_params=pltpu.CompilerParams(
            dimension_semantics=("parallel",),
        ),
    )(hidden_states.astype(jnp.float32), W, b)
